# Initial kernel scaffold; baseline (speedup 1.0000x reference)
import jax
import jax.numpy as jnp
from jax.experimental import pallas as pl


def kernel(user_table, item_table, theta, xpath_indices, pos_indices, pos_values, neg_indices, neg_values):
    raise NotImplementedError("write your pallas kernel here")


# speedup vs baseline: 3.9203x; 3.9203x over previous
import functools

import jax
import jax.numpy as jnp
from jax import lax
from jax.experimental import pallas as pl
from jax.experimental.pallas import tpu as pltpu
from jax.experimental.pallas import tpu_sc as plsc

N_USERS = 5000
N_ITEMS = 5000
N = N_USERS + N_ITEMS
D = 256
HALF = 128
N_LAYERS = 3
ALPHA = 0.5
N_PATHS = 6
E_PATH = 80000
E_GRAPH = 160000
EP = N_PATHS * E_PATH
EG = 2 * E_GRAPH

NPAD = 10240
NTILES = 16
EP_T = EP // NTILES
EG_T = EG // NTILES
CHUNK = 400
BATCH = 80
NB = CHUNK // BATCH
NG = BATCH // 16
ROWS_T = NPAD // NTILES
RCH = 80
NRC = ROWS_T // RCH

_STAGE = 5


def _recip(x):
    xi = lax.bitcast_convert_type(x, jnp.int32)
    r = lax.bitcast_convert_type(jnp.int32(0x7EF311C3) - xi, jnp.float32)
    for _ in range(4):
        r = r * (2.0 - x * r)
    return r


def _sc_body(x_hbm, prow, pcol, pval, grow, gcol, gval,
             out_hbm, ea_hbm, eb_hbm,
             acc_sh, h_all, rows2, cols2, rows_v, cols_v, vals_v, gbuf,
             wbuf, hist, h16c, invh):
    c = lax.axis_index("c")
    s = lax.axis_index("s")
    col_off = c * NPAD
    row_base = s * ROWS_T
    sf = jnp.where(s < 8, jnp.float32(1.0), jnp.float32(-ALPHA))
    z16 = jnp.zeros((16,), jnp.float32)
    m0 = jnp.where(lax.iota(jnp.int32, 16) == 0,
                   jnp.float32(1.0), jnp.float32(0.0))

    def edge_pass(src_hbm, row_hbm, col_hbm, val_hbm, tile_edges, is_p):
        nch = tile_edges // CHUNK

        def chunk_body(ch, _):
            e0 = s * tile_edges + ch * CHUNK
            pltpu.sync_copy(row_hbm.at[pl.ds(e0, CHUNK)], rows_v)
            pltpu.sync_copy(col_hbm.at[pl.ds(e0, CHUNK)], cols_v)
            pltpu.sync_copy(val_hbm.at[pl.ds(e0, CHUNK)], vals_v)

            def prep(i, _):
                vv = vals_v[pl.ds(i * 16, 16)]
                if is_p:
                    vals_v[pl.ds(i * 16, 16)] = jnp.exp(vv)
                else:
                    vals_v[pl.ds(i * 16, 16)] = vv * sf
                return 0

            lax.fori_loop(0, CHUNK // 16, prep, 0)

            if is_p:
                def shist(g, _):
                    rv = rows_v[pl.ds(g * 16, 16)]
                    vv = vals_v[pl.ds(g * 16, 16)]
                    for l in range(16):
                        r = rv[l]
                        lane = jnp.full((16,), l, jnp.int32)
                        c16 = vv.at[lane].get(mode="promise_in_bounds") * m0
                        hist[pl.ds(r, 16)] = hist[pl.ds(r, 16)] + c16
                    return 0

                lax.fori_loop(0, CHUNK // 16, shist, 0)

            def r2copy(b, _):
                for q in range(NG):
                    rows2[b, pl.ds(q * 16, 16)] = (
                        rows_v[pl.ds(b * BATCH + q * 16, 16)])
                    cols2[b, pl.ds(q * 16, 16)] = (
                        cols_v[pl.ds(b * BATCH + q * 16, 16)] + col_off)
                return 0

            lax.fori_loop(0, NB, r2copy, 0)

            def batch_body(b, _):
                pltpu.sync_copy(src_hbm.at[cols2.at[b]], gbuf)

                def group_body(g, _):
                    v16 = vals_v[pl.ds(b * BATCH + g * 16, 16)]
                    base = g * 16
                    for e in range(16):
                        lane = jnp.full((16,), e, jnp.int32)
                        bc = v16.at[lane].get(mode="promise_in_bounds")
                        for j in range(8):
                            gbuf[base + e, pl.ds(j * 16, 16)] = (
                                gbuf[base + e, pl.ds(j * 16, 16)] * bc)
                    return 0

                lax.fori_loop(0, NG, group_body, 0)
                pltpu.sync_copy(gbuf, acc_sh.at[rows2.at[b]], add=True)
                return 0

            lax.fori_loop(0, NB, batch_body, 0)
            return 0

        lax.fori_loop(0, nch, chunk_body, 0)

    def wb_p():
        def hred(j, _):
            pltpu.sync_copy(h_all.at[:, pl.ds(row_base + j * 128, 128)], h16c)

            def redq(q, _):
                tot = h16c[0, pl.ds(q * 16, 16)]
                for t in range(1, NTILES):
                    tot = tot + h16c[t, pl.ds(q * 16, 16)]
                invh[pl.ds(j * 128 + q * 16, 16)] = _recip(tot + 1e-8)
                return 0

            lax.fori_loop(0, 8, redq, 0)
            return 0

        lax.fori_loop(0, ROWS_T // 128, hred, 0)

        def k_body(k, _):
            r0 = row_base + k * RCH
            gr0 = col_off + r0
            pltpu.sync_copy(acc_sh.at[pl.ds(r0, RCH)], wbuf)

            def rfix(r, _):
                idx = k * RCH + r
                base16 = (idx // 16) * 16
                lanev = jnp.full((16,), idx - base16, jnp.int32)
                v16 = invh[pl.ds(base16, 16)]
                inv = v16.at[lanev].get(mode="promise_in_bounds")
                for j in range(8):
                    wbuf[r, pl.ds(j * 16, 16)] = (
                        wbuf[r, pl.ds(j * 16, 16)] * inv)
                return 0

            lax.fori_loop(0, RCH, rfix, 0)
            pltpu.sync_copy(wbuf, ea_hbm.at[pl.ds(gr0, RCH)])
            pltpu.sync_copy(wbuf, out_hbm.at[pl.ds(gr0, RCH)])
            return 0

        lax.fori_loop(0, NRC, k_body, 0)

    def init_layer(eprev):
        def k_body(k, _):
            r0 = row_base + k * RCH
            gr0 = col_off + r0
            pltpu.sync_copy(eprev.at[pl.ds(gr0, RCH)], wbuf)

            def rsc(r, _):
                for j in range(8):
                    wbuf[r, pl.ds(j * 16, 16)] = (
                        wbuf[r, pl.ds(j * 16, 16)] * ALPHA)
                return 0

            lax.fori_loop(0, RCH, rsc, 0)
            pltpu.sync_copy(wbuf, acc_sh.at[pl.ds(r0, RCH)])
            return 0

        lax.fori_loop(0, NRC, k_body, 0)

    def wb_layer(enext, last):
        def k_body(k, _):
            r0 = row_base + k * RCH
            gr0 = col_off + r0
            pltpu.sync_copy(acc_sh.at[pl.ds(r0, RCH)], wbuf)
            pltpu.sync_copy(out_hbm.at[pl.ds(gr0, RCH)], gbuf)

            def racc(r, _):
                for j in range(8):
                    t = (gbuf[r, pl.ds(j * 16, 16)]
                         + wbuf[r, pl.ds(j * 16, 16)])
                    if last:
                        t = t * 0.25
                    gbuf[r, pl.ds(j * 16, 16)] = t
                return 0

            lax.fori_loop(0, RCH, racc, 0)
            pltpu.sync_copy(gbuf, out_hbm.at[pl.ds(gr0, RCH)])
            if enext is not None:
                pltpu.sync_copy(wbuf, enext.at[pl.ds(gr0, RCH)])
            return 0

        lax.fori_loop(0, NRC, k_body, 0)

    def zw(r, _):
        for j in range(8):
            wbuf[r, pl.ds(j * 16, 16)] = z16
        return 0

    lax.fori_loop(0, RCH, zw, 0)

    def zhist(i, _):
        hist[pl.ds(i * 16, 16)] = z16
        return 0

    lax.fori_loop(0, NPAD // 16, zhist, 0)

    if _STAGE == 0:
        pltpu.sync_copy(wbuf, out_hbm.at[pl.ds(col_off + row_base, RCH)])
        return

    def zacc(k, _):
        pltpu.sync_copy(wbuf, acc_sh.at[pl.ds(row_base + k * RCH, RCH)])
        return 0

    lax.fori_loop(0, NRC, zacc, 0)
    plsc.subcore_barrier()

    if _STAGE == 1:
        pltpu.sync_copy(wbuf, out_hbm.at[pl.ds(col_off + row_base, RCH)])
        return

    if _STAGE >= 4:
        edge_pass(x_hbm, prow, pcol, pval, EP_T, True)
    pltpu.sync_copy(hist, h_all.at[s])
    plsc.subcore_barrier()
    wb_p()

    if _STAGE < 5:
        return

    init_layer(ea_hbm)
    plsc.subcore_barrier()
    edge_pass(ea_hbm, grow, gcol, gval, EG_T, False)
    plsc.subcore_barrier()
    wb_layer(eb_hbm, False)

    init_layer(eb_hbm)
    plsc.subcore_barrier()
    edge_pass(eb_hbm, grow, gcol, gval, EG_T, False)
    plsc.subcore_barrier()
    wb_layer(ea_hbm, False)

    init_layer(ea_hbm)
    plsc.subcore_barrier()
    edge_pass(ea_hbm, grow, gcol, gval, EG_T, False)
    plsc.subcore_barrier()
    wb_layer(None, True)


_sc_call = functools.partial(
    pl.kernel,
    mesh=plsc.VectorSubcoreMesh(core_axis_name="c", subcore_axis_name="s"),
    out_type=[
        jax.ShapeDtypeStruct((2 * NPAD, HALF), jnp.float32),
        jax.ShapeDtypeStruct((2 * NPAD, HALF), jnp.float32),
        jax.ShapeDtypeStruct((2 * NPAD, HALF), jnp.float32),
    ],
    scratch_types=[
        pltpu.VMEM_SHARED((NPAD, HALF), jnp.float32),
        pltpu.VMEM_SHARED((NTILES, NPAD), jnp.float32),
        pltpu.VMEM((NB, BATCH), jnp.int32),
        pltpu.VMEM((NB, BATCH), jnp.int32),
        pltpu.VMEM((CHUNK,), jnp.int32),
        pltpu.VMEM((CHUNK,), jnp.int32),
        pltpu.VMEM((CHUNK,), jnp.float32),
        pltpu.VMEM((BATCH, HALF), jnp.float32),
        pltpu.VMEM((RCH, HALF), jnp.float32),
        pltpu.VMEM((NPAD,), jnp.float32),
        pltpu.VMEM((NTILES, 128), jnp.float32),
        pltpu.VMEM((ROWS_T,), jnp.float32),
    ],
)(_sc_body)


def kernel(user_table, item_table, theta, xpath_indices, pos_indices,
           pos_values, neg_indices, neg_values):
    all_emb = jnp.concatenate([user_table, item_table], axis=0)
    pad = ((0, NPAD - N), (0, 0))
    x_split = jnp.concatenate([jnp.pad(all_emb[:, :HALF], pad),
                               jnp.pad(all_emb[:, HALF:], pad)], axis=0)

    prow = xpath_indices[:, 0, :].reshape(EP)
    pcol = xpath_indices[:, 1, :].reshape(EP)
    pval = jnp.repeat(theta, E_PATH)

    grow = jnp.concatenate([pos_indices[0], neg_indices[0]])
    gcol = jnp.concatenate([pos_indices[1], neg_indices[1]])
    gval = jnp.concatenate([pos_values, neg_values])

    out, _ea, _eb = _sc_call(x_split, prow, pcol, pval, grow, gcol, gval)
    light = jnp.concatenate([out[:N], out[NPAD:NPAD + N]], axis=1)
    return light[:N_USERS], light[N_USERS:]
